# Initial kernel scaffold; baseline (speedup 1.0000x reference)
import functools

import jax
import jax.numpy as jnp
from jax import lax
from jax.experimental import pallas as pl
from jax.experimental.pallas import tpu as pltpu
from jax.experimental.pallas import tpu_sc as plsc

N = 10000
E = 320000
D = 128
R = 20
B = 20
EPS = 1e-5

NC, NS = 2, 16
NW = NC * NS
EPW = E // NW
G = 80
CH = EPW // G
NPS = N // NS


def _transform_body(comp_ref, h_ref, v_ref, y_ref):
    r = pl.program_id(0)
    w = jnp.zeros((D, D), jnp.float32)
    for b in range(B):
        w = w + comp_ref[r, b] * v_ref[b]
    y_ref[...] = jnp.dot(h_ref[...], w, preferred_element_type=jnp.float32)


def _transform(comp, h, v):
    return pl.pallas_call(
        _transform_body,
        grid=(R,),
        in_specs=[
            pl.BlockSpec(memory_space=pltpu.SMEM),
            pl.BlockSpec((N, D), lambda r: (0, 0)),
            pl.BlockSpec((B, D, D), lambda r: (0, 0, 0)),
        ],
        out_specs=pl.BlockSpec((N, D), lambda r: (r, 0)),
        out_shape=jax.ShapeDtypeStruct((R * N, D), jnp.float32),
    )(comp, h, v)


_MESH = plsc.VectorSubcoreMesh(
    core_axis_name="c", subcore_axis_name="s", num_cores=NC, num_subcores=NS
)


@functools.partial(
    pl.kernel,
    out_type=jax.ShapeDtypeStruct((NC, N, D), jnp.float32),
    mesh=_MESH,
    scratch_types=[
        pltpu.VMEM((CH, G), jnp.int32),
        pltpu.VMEM((CH, G), jnp.int32),
        pltpu.VMEM((G, D), jnp.float32),
        pltpu.VMEM_SHARED((N, D), jnp.float32),
        pltpu.SemaphoreType.DMA,
    ],
)
def _scatter_accum(y_hbm, gidx_hbm, dst_hbm, zeros_hbm, out_hbm,
                   idx_v, dst_v, rows_v, msg_sh, sem):
    cid = lax.axis_index("c")
    sid = lax.axis_index("s")
    wid = sid * NC + cid

    row0 = sid * NPS
    pltpu.sync_copy(zeros_hbm.at[pl.ds(row0, NPS)], msg_sh.at[pl.ds(row0, NPS)])

    c0 = wid * CH
    pltpu.sync_copy(gidx_hbm.at[pl.ds(c0, CH)], idx_v)
    pltpu.sync_copy(dst_hbm.at[pl.ds(c0, CH)], dst_v)
    plsc.subcore_barrier()

    def body(i, carry):
        pltpu.async_copy(y_hbm.at[idx_v.at[i]], rows_v, sem).wait()
        pltpu.sync_copy(rows_v, msg_sh.at[dst_v.at[i]], add=True)
        return carry

    lax.fori_loop(0, CH, body, 0)

    plsc.subcore_barrier()
    pltpu.sync_copy(msg_sh.at[pl.ds(row0, NPS)],
                    out_hbm.at[cid, pl.ds(row0, NPS)])


def _bn_body(msg_ref, h_ref, loop_ref, bias_ref, gamma_ref, beta_ref, o_ref,
             *, relu):
    z = (msg_ref[0] + msg_ref[1] + bias_ref[...]
         + jnp.dot(h_ref[...], loop_ref[...], preferred_element_type=jnp.float32))
    mu = jnp.mean(z, axis=0, keepdims=True)
    d = z - mu
    var = jnp.mean(d * d, axis=0, keepdims=True)
    o = d * lax.rsqrt(var + EPS) * gamma_ref[...] + beta_ref[...]
    if relu:
        o = jnp.maximum(o, 0.0)
    o_ref[...] = o


def _bn(msg2, h, loop_w, bias, gamma, beta, relu):
    return pl.pallas_call(
        functools.partial(_bn_body, relu=relu),
        out_shape=jax.ShapeDtypeStruct((N, D), jnp.float32),
    )(msg2, h, loop_w, bias.reshape(1, D), gamma.reshape(1, D),
      beta.reshape(1, D))


def kernel(x, edge_index, edge_type,
           V0, comp0, loop0, bias0, gamma0, beta0,
           V1, comp1, loop1, bias1, gamma1, beta1,
           V2, comp2, loop2, bias2, gamma2, beta2):
    src = edge_index[0]
    dst = edge_index[1]
    gidx = (edge_type * N + src).reshape(E // G, G)
    dst2 = dst.reshape(E // G, G)
    zeros = jnp.zeros((N, D), jnp.float32)

    params = [
        (V0, comp0, loop0, bias0, gamma0, beta0),
        (V1, comp1, loop1, bias1, gamma1, beta1),
        (V2, comp2, loop2, bias2, gamma2, beta2),
    ]
    h = x
    for i, (V, comp, loop_w, bias, gamma, beta) in enumerate(params):
        y = _transform(comp, h, V)
        msg2 = _scatter_accum(y, gidx, dst2, zeros)
        h = _bn(msg2, h, loop_w, bias, gamma, beta, relu=(i == 2))
    return h


# speedup vs baseline: 5.1651x; 5.1651x over previous
import functools

import jax
import jax.numpy as jnp
from jax import lax
from jax.experimental import pallas as pl
from jax.experimental.pallas import tpu as pltpu
from jax.experimental.pallas import tpu_sc as plsc

N = 10000
E = 320000
D = 128
R = 20
B = 20
EPS = 1e-5

NC, NS = 2, 16
NW = NC * NS
EPW = E // NW
G = 128
CH = 79
NPS = 632
NPAD = NPS * NS


def _transform_body(comp_ref, h_ref, v_ref, y_ref):
    r = pl.program_id(0)
    w = jnp.zeros((D, D), jnp.float32)
    for b in range(B):
        w = w + comp_ref[r, b] * v_ref[b]
    y_ref[...] = jnp.dot(h_ref[...], w, preferred_element_type=jnp.float32)


def _transform(comp, h, v):
    return pl.pallas_call(
        _transform_body,
        grid=(R,),
        in_specs=[
            pl.BlockSpec(memory_space=pltpu.SMEM),
            pl.BlockSpec((N, D), lambda r: (0, 0)),
            pl.BlockSpec((B, D, D), lambda r: (0, 0, 0)),
        ],
        out_specs=pl.BlockSpec((N, D), lambda r: (r, 0)),
        out_shape=jax.ShapeDtypeStruct((R * N, D), jnp.float32),
    )(comp, h, v)


_MESH = plsc.VectorSubcoreMesh(
    core_axis_name="c", subcore_axis_name="s", num_cores=NC, num_subcores=NS
)


@functools.partial(
    pl.kernel,
    out_type=jax.ShapeDtypeStruct((NC, NPAD, D), jnp.float32),
    mesh=_MESH,
    scratch_types=[
        pltpu.VMEM((CH, G), jnp.int32),
        pltpu.VMEM((CH, G), jnp.int32),
        pltpu.VMEM((G, D), jnp.float32),
        pltpu.VMEM_SHARED((NPAD, D), jnp.float32),
        pltpu.SemaphoreType.DMA,
    ],
)
def _scatter_accum(y_hbm, gidx_hbm, dst_hbm, zeros_hbm, out_hbm,
                   idx_v, dst_v, rows_v, msg_sh, sem):
    cid = lax.axis_index("c")
    sid = lax.axis_index("s")
    wid = sid * NC + cid

    row0 = sid * NPS
    pltpu.sync_copy(zeros_hbm.at[pl.ds(row0, NPS)], msg_sh.at[pl.ds(row0, NPS)])

    pltpu.sync_copy(gidx_hbm.at[wid], idx_v)
    pltpu.sync_copy(dst_hbm.at[wid], dst_v)
    plsc.subcore_barrier()

    def body(i, carry):
        pltpu.async_copy(y_hbm.at[idx_v.at[i]], rows_v, sem).wait()
        pltpu.sync_copy(rows_v, msg_sh.at[dst_v.at[i]], add=True)
        return carry

    lax.fori_loop(0, CH, body, 0)

    plsc.subcore_barrier()
    pltpu.sync_copy(msg_sh.at[pl.ds(row0, NPS)],
                    out_hbm.at[cid, pl.ds(row0, NPS)])


def _bn_body(msg_ref, h_ref, loop_ref, bias_ref, gamma_ref, beta_ref, o_ref,
             *, relu):
    z = (msg_ref[0, :N] + msg_ref[1, :N] + bias_ref[...]
         + jnp.dot(h_ref[...], loop_ref[...], preferred_element_type=jnp.float32))
    mu = jnp.mean(z, axis=0, keepdims=True)
    d = z - mu
    var = jnp.mean(d * d, axis=0, keepdims=True)
    o = d * lax.rsqrt(var + EPS) * gamma_ref[...] + beta_ref[...]
    if relu:
        o = jnp.maximum(o, 0.0)
    o_ref[...] = o


def _bn(msg2, h, loop_w, bias, gamma, beta, relu):
    return pl.pallas_call(
        functools.partial(_bn_body, relu=relu),
        out_shape=jax.ShapeDtypeStruct((N, D), jnp.float32),
    )(msg2, h, loop_w, bias.reshape(1, D), gamma.reshape(1, D),
      beta.reshape(1, D))


def kernel(x, edge_index, edge_type,
           V0, comp0, loop0, bias0, gamma0, beta0,
           V1, comp1, loop1, bias1, gamma1, beta1,
           V2, comp2, loop2, bias2, gamma2, beta2):
    src = edge_index[0]
    dst = edge_index[1]
    pad = CH * G - EPW
    gidx = jnp.pad((edge_type * N + src).reshape(NW, EPW),
                   ((0, 0), (0, pad))).reshape(NW, CH, G)
    dst2 = jnp.pad(dst.reshape(NW, EPW), ((0, 0), (0, pad)),
                   constant_values=N).reshape(NW, CH, G)
    zeros = jnp.zeros((NPAD, D), jnp.float32)

    params = [
        (V0, comp0, loop0, bias0, gamma0, beta0),
        (V1, comp1, loop1, bias1, gamma1, beta1),
        (V2, comp2, loop2, bias2, gamma2, beta2),
    ]
    h = x
    for i, (V, comp, loop_w, bias, gamma, beta) in enumerate(params):
        y = _transform(comp, h, V)
        msg2 = _scatter_accum(y, gidx, dst2, zeros)
        h = _bn(msg2, h, loop_w, bias, gamma, beta, relu=(i == 2))
    return h
